# baseline (speedup 1.0000x reference)
import functools

import jax
import jax.numpy as jnp
from jax import lax
from jax.experimental import pallas as pl
from jax.experimental.pallas import tpu as pltpu
from jax.experimental.pallas import tpu_sc as plsc

_TB = 512
_RB = 256
_T = 2048
_D = 1024
_E = 8
_RSLOTS = 2 * _T + _E * _RB
_NRB = _RSLOTS // _RB
_NW = 32


def _silu(v):
    return v * jax.lax.logistic(v)


def _route_body(x_ref, gate_ref, s0_ref, s1_ref, w0_ref, w1_ref, be_ref,
                act_ref, *, rb):
    x = x_ref[...]
    t = x.shape[0]
    logits = jnp.dot(x, gate_ref[...].T, preferred_element_type=jnp.float32)
    m = jnp.max(logits, axis=1, keepdims=True)
    p = jnp.exp(logits - m)
    s = p / jnp.sum(p, axis=1, keepdims=True)
    n_e = s.shape[1]
    lane = lax.broadcasted_iota(jnp.int32, s.shape, 1)
    m1 = jnp.max(s, axis=1, keepdims=True)
    i1 = jnp.min(jnp.where(s == m1, lane, n_e), axis=1, keepdims=True)
    sel1 = lane == i1
    s2 = jnp.where(sel1, -jnp.inf, s)
    m2 = jnp.max(s2, axis=1, keepdims=True)
    i2 = jnp.min(jnp.where(s2 == m2, lane, n_e), axis=1, keepdims=True)
    sel2 = lane == i2
    denom = m1 + m2 + 1e-20
    sel01 = (sel1 | sel2).astype(jnp.float32)

    nb = t // rb
    r_i = lax.broadcasted_iota(jnp.int32, (rb, rb), 0)
    c_i = lax.broadcasted_iota(jnp.int32, (rb, rb), 1)
    tril = (c_i < r_i).astype(jnp.float32)
    blocks = []
    prefix = jnp.zeros((1, n_e), jnp.float32)
    for b in range(nb):
        sb = sel01[b * rb:(b + 1) * rb, :]
        blocks.append(jnp.dot(tril, sb, preferred_element_type=jnp.float32)
                      + prefix)
        prefix = prefix + jnp.sum(sb, axis=0, keepdims=True)
    cnt = jnp.concatenate(blocks, axis=0)
    counts_i = prefix.astype(jnp.int32)
    padded_i = ((counts_i + rb - 1) // rb) * rb
    padded_f = padded_i.astype(jnp.float32)
    triu = (lax.broadcasted_iota(jnp.int32, (n_e, n_e), 0)
            < lax.broadcasted_iota(jnp.int32, (n_e, n_e), 1)).astype(jnp.float32)
    aligned_f = jnp.dot(padded_f, triu, preferred_element_type=jnp.float32)
    base = aligned_f + cnt
    s0_ref[...] = jnp.sum(jnp.where(sel1, base, 0.0), axis=1,
                          keepdims=True).astype(jnp.int32)
    s1_ref[...] = jnp.sum(jnp.where(sel2, base, 0.0), axis=1,
                          keepdims=True).astype(jnp.int32)
    w0_ref[...] = m1 / denom
    w1_ref[...] = m2 / denom

    nbr = be_ref.shape[0]
    srow = lax.broadcasted_iota(jnp.int32, (nbr, n_e), 0) * rb
    al_b = jnp.broadcast_to(aligned_f.astype(jnp.int32), (nbr, n_e))
    pad_b = jnp.broadcast_to(padded_i, (nbr, n_e))
    eidx = lax.broadcasted_iota(jnp.int32, (nbr, n_e), 1)
    mask = (al_b <= srow) & (srow < al_b + pad_b)
    be_ref[...] = jnp.sum(jnp.where(mask, eidx, 0), axis=1, keepdims=True)
    act_ref[...] = jnp.sum(mask.astype(jnp.int32), axis=1, keepdims=True)


def _shared_body(x_ref, sg_ref, su_ref, sd_ref, out_ref):
    x = x_ref[...]
    g = jnp.dot(x, sg_ref[...].T, preferred_element_type=jnp.float32)
    u = jnp.dot(x, su_ref[...].T, preferred_element_type=jnp.float32)
    h = _silu(g) * u
    out_ref[...] = jnp.dot(h, sd_ref[...].T, preferred_element_type=jnp.float32)


def _ragged_body(be_ref, act_ref, xs_ref, w1_ref, w3_ref, w2_ref, y_ref):
    i = pl.program_id(0)

    @pl.when(act_ref[i] != 0)
    def _():
        xb = xs_ref[...]
        g = jnp.dot(xb, w1_ref[0].T, preferred_element_type=jnp.float32)
        u = jnp.dot(xb, w3_ref[0].T, preferred_element_type=jnp.float32)
        h = _silu(g) * u
        y_ref[...] = jnp.dot(h, w2_ref[0].T,
                             preferred_element_type=jnp.float32)


def _dispatch_sc(x, slot0, slot1):
    tpw = _T // _NW
    mesh = plsc.VectorSubcoreMesh(core_axis_name="c", subcore_axis_name="s")

    @functools.partial(
        pl.kernel,
        out_type=jax.ShapeDtypeStruct((_RSLOTS, _D), jnp.float32),
        mesh=mesh,
        compiler_params=pltpu.CompilerParams(needs_layout_passes=False),
        scratch_types=[
            pltpu.VMEM((tpw,), jnp.int32),
            pltpu.VMEM((tpw,), jnp.int32),
            pltpu.VMEM((tpw, _D), jnp.float32),
            pltpu.SemaphoreType.DMA,
            pltpu.SemaphoreType.DMA,
        ],
    )
    def k(s0_hbm, s1_hbm, x_hbm, xs_hbm, idx0_v, idx1_v, rows_v, sem0, sem1):
        wid = lax.axis_index("s") * 2 + lax.axis_index("c")
        t0 = wid * tpw
        pltpu.sync_copy(s0_hbm.at[pl.ds(t0, tpw)], idx0_v)
        pltpu.sync_copy(s1_hbm.at[pl.ds(t0, tpw)], idx1_v)
        pltpu.sync_copy(x_hbm.at[pl.ds(t0, tpw)], rows_v)
        c0 = pltpu.async_copy(rows_v, xs_hbm.at[idx0_v], sem0)
        c1 = pltpu.async_copy(rows_v, xs_hbm.at[idx1_v], sem1)
        c0.wait()
        c1.wait()

    return k(slot0, slot1, x)


def _combine_sc(slot0, slot1, w0a, w1a, y_sorted, shared):
    tpw = _T // _NW
    ck = 16
    mesh = plsc.VectorSubcoreMesh(core_axis_name="c", subcore_axis_name="s")

    @functools.partial(
        pl.kernel,
        out_type=jax.ShapeDtypeStruct((_T, _D), jnp.float32),
        mesh=mesh,
        compiler_params=pltpu.CompilerParams(needs_layout_passes=False),
        scratch_types=[
            pltpu.VMEM((tpw,), jnp.int32),
            pltpu.VMEM((tpw,), jnp.int32),
            pltpu.VMEM((tpw,), jnp.float32),
            pltpu.VMEM((tpw,), jnp.float32),
            pltpu.VMEM((ck, _D), jnp.float32),
            pltpu.VMEM((ck, _D), jnp.float32),
            pltpu.VMEM((ck, _D), jnp.float32),
            pltpu.VMEM((ck, _D), jnp.float32),
            pltpu.VMEM((ck, _D), jnp.float32),
            pltpu.VMEM((ck, _D), jnp.float32),
            pltpu.SemaphoreType.DMA,
            pltpu.SemaphoreType.DMA,
            pltpu.SemaphoreType.DMA,
            pltpu.SemaphoreType.DMA,
            pltpu.SemaphoreType.DMA,
            pltpu.SemaphoreType.DMA,
            pltpu.SemaphoreType.DMA,
            pltpu.SemaphoreType.DMA,
        ],
    )
    def k(s0_hbm, s1_hbm, w0_hbm, w1_hbm, y_hbm, sh_hbm, out_hbm,
          s0_v, s1_v, w0_v, w1_v, r0a_v, r1a_v, r0b_v, r1b_v, acca_v, accb_v,
          sa0, sa1, sb0, sb1, sha, shb, soa, sob):
        wid = lax.axis_index("s") * 2 + lax.axis_index("c")
        t0 = wid * tpw
        zeros = jnp.zeros((16,), jnp.int32)
        pltpu.sync_copy(s0_hbm.at[pl.ds(t0, tpw)], s0_v)
        pltpu.sync_copy(s1_hbm.at[pl.ds(t0, tpw)], s1_v)
        pltpu.sync_copy(w0_hbm.at[pl.ds(t0, tpw)], w0_v)
        pltpu.sync_copy(w1_hbm.at[pl.ds(t0, tpw)], w1_v)
        bufs = ((r0a_v, r1a_v, sa0, sa1), (r0b_v, r1b_v, sb0, sb1))
        accs = ((acca_v, sha, soa), (accb_v, shb, sob))
        nch = tpw // ck

        def fire(chn):
            r0, r1, g0, g1 = bufs[chn % 2]
            d0 = pltpu.async_copy(y_hbm.at[s0_v.at[pl.ds(chn * ck, ck)]],
                                  r0, g0)
            d1 = pltpu.async_copy(y_hbm.at[s1_v.at[pl.ds(chn * ck, ck)]],
                                  r1, g1)
            return d0, d1

        def fire_sh(chn):
            acc, ssem, _ = accs[chn % 2]
            return pltpu.async_copy(sh_hbm.at[pl.ds(t0 + chn * ck, ck)],
                                    acc, ssem)

        pend = fire(0)
        pend_sh = [fire_sh(0), None]
        pend_out = [None, None]
        for chn in range(nch):
            sl2 = (chn + 1) % 2
            if chn + 1 < nch:
                if pend_out[sl2] is not None:
                    pend_out[sl2].wait()
                    pend_out[sl2] = None
                nxt = fire(chn + 1)
                pend_sh[sl2] = fire_sh(chn + 1)
            else:
                nxt = None
            pend[0].wait()
            pend[1].wait()
            pend_sh[chn % 2].wait()
            r0, r1 = bufs[chn % 2][0], bufs[chn % 2][1]
            acc_v = accs[chn % 2][0]
            tb = t0 + chn * ck

            def _tok(i, carry, *, _chn=chn, _r0=r0, _r1=r1, _acc=acc_v):
                w0 = plsc.load_gather(w0_v, [zeros + (_chn * ck + i)])
                w1v = plsc.load_gather(w1_v, [zeros + (_chn * ck + i)])
                for j in range(_D // 16):
                    sl = pl.ds(j * 16, 16)
                    _acc[i, sl] = (_acc[i, sl] + w0 * _r0[i, sl]
                                   + w1v * _r1[i, sl])
                return carry
            lax.fori_loop(0, ck, _tok, 0)
            pend_out[chn % 2] = pltpu.async_copy(
                acc_v, out_hbm.at[pl.ds(tb, ck)], accs[chn % 2][2])
            pend = nxt
        for po in pend_out:
            if po is not None:
                po.wait()

    return k(slot0, slot1, w0a, w1a, y_sorted, shared)


def kernel(hidden_states, gate_w, w1, w2, w3, sg, su, sd):
    b, s, d = hidden_states.shape
    x = hidden_states.reshape(-1, d).astype(jnp.float32)
    t = x.shape[0]
    n_exp, d_ff, _ = w1.shape
    sf = sg.shape[0]
    n_tb = t // _TB

    slot0, slot1, w0a, w1a, be, act = pl.pallas_call(
        functools.partial(_route_body, rb=_RB),
        in_specs=[
            pl.BlockSpec((t, d), lambda: (0, 0)),
            pl.BlockSpec(gate_w.shape, lambda: (0, 0)),
        ],
        out_specs=[
            pl.BlockSpec((t, 1), lambda: (0, 0)),
            pl.BlockSpec((t, 1), lambda: (0, 0)),
            pl.BlockSpec((t, 1), lambda: (0, 0)),
            pl.BlockSpec((t, 1), lambda: (0, 0)),
            pl.BlockSpec((_NRB, 1), lambda: (0, 0)),
            pl.BlockSpec((_NRB, 1), lambda: (0, 0)),
        ],
        out_shape=[
            jax.ShapeDtypeStruct((t, 1), jnp.int32),
            jax.ShapeDtypeStruct((t, 1), jnp.int32),
            jax.ShapeDtypeStruct((t, 1), jnp.float32),
            jax.ShapeDtypeStruct((t, 1), jnp.float32),
            jax.ShapeDtypeStruct((_NRB, 1), jnp.int32),
            jax.ShapeDtypeStruct((_NRB, 1), jnp.int32),
        ],
    )(x, gate_w)

    shared = pl.pallas_call(
        _shared_body,
        grid=(n_tb,),
        in_specs=[
            pl.BlockSpec((_TB, d), lambda i: (i, 0)),
            pl.BlockSpec((sf, d), lambda i: (0, 0)),
            pl.BlockSpec((sf, d), lambda i: (0, 0)),
            pl.BlockSpec((d, sf), lambda i: (0, 0)),
        ],
        out_specs=pl.BlockSpec((_TB, d), lambda i: (i, 0)),
        out_shape=jax.ShapeDtypeStruct((t, d), jnp.float32),
    )(x, sg, su, sd)

    slot0 = slot0.reshape(-1)
    slot1 = slot1.reshape(-1)
    w0a = w0a.reshape(-1)
    w1a = w1a.reshape(-1)
    be_flat = be.reshape(-1)

    x_sorted = _dispatch_sc(x, slot0, slot1)

    y_sorted = pl.pallas_call(
        _ragged_body,
        grid_spec=pltpu.PrefetchScalarGridSpec(
            num_scalar_prefetch=2,
            grid=(_NRB,),
            in_specs=[
                pl.BlockSpec((_RB, d), lambda i, be_r, a_r: (i, 0)),
                pl.BlockSpec((1, d_ff, d),
                             lambda i, be_r, a_r: (be_r[i], 0, 0)),
                pl.BlockSpec((1, d_ff, d),
                             lambda i, be_r, a_r: (be_r[i], 0, 0)),
                pl.BlockSpec((1, d, d_ff),
                             lambda i, be_r, a_r: (be_r[i], 0, 0)),
            ],
            out_specs=pl.BlockSpec((_RB, d), lambda i, be_r, a_r: (i, 0)),
        ),
        out_shape=jax.ShapeDtypeStruct((_RSLOTS, d), jnp.float32),
    )(be_flat, act.reshape(-1), x_sorted, w1, w3, w2)

    out = _combine_sc(slot0, slot1, w0a, w1a, y_sorted, shared)
    return out.reshape(b, s, d).astype(hidden_states.dtype)


# speedup vs baseline: 1.0099x; 1.0099x over previous
import functools

import jax
import jax.numpy as jnp
from jax import lax
from jax.experimental import pallas as pl
from jax.experimental.pallas import tpu as pltpu
from jax.experimental.pallas import tpu_sc as plsc

_TB = 512
_RB = 256
_T = 2048
_D = 1024
_E = 8
_RSLOTS = 2 * _T + _E * _RB
_NRB = _RSLOTS // _RB
_NW = 32


def _silu(v):
    return v * jax.lax.logistic(v)


def _route_body(x_ref, gate_ref, s0_ref, s1_ref, w0_ref, w1_ref, be_ref,
                act_ref, *, rb):
    x = x_ref[...]
    t = x.shape[0]
    logits = jnp.dot(x, gate_ref[...].T, preferred_element_type=jnp.float32)
    m = jnp.max(logits, axis=1, keepdims=True)
    p = jnp.exp(logits - m)
    s = p / jnp.sum(p, axis=1, keepdims=True)
    n_e = s.shape[1]
    lane = lax.broadcasted_iota(jnp.int32, s.shape, 1)
    m1 = jnp.max(s, axis=1, keepdims=True)
    i1 = jnp.min(jnp.where(s == m1, lane, n_e), axis=1, keepdims=True)
    sel1 = lane == i1
    s2 = jnp.where(sel1, -jnp.inf, s)
    m2 = jnp.max(s2, axis=1, keepdims=True)
    i2 = jnp.min(jnp.where(s2 == m2, lane, n_e), axis=1, keepdims=True)
    sel2 = lane == i2
    denom = m1 + m2 + 1e-20
    sel01 = (sel1 | sel2).astype(jnp.float32)

    nb = t // rb
    r_i = lax.broadcasted_iota(jnp.int32, (rb, rb), 0)
    c_i = lax.broadcasted_iota(jnp.int32, (rb, rb), 1)
    tril = (c_i < r_i).astype(jnp.float32)
    blocks = []
    prefix = jnp.zeros((1, n_e), jnp.float32)
    for b in range(nb):
        sb = sel01[b * rb:(b + 1) * rb, :]
        blocks.append(jnp.dot(tril, sb, preferred_element_type=jnp.float32)
                      + prefix)
        prefix = prefix + jnp.sum(sb, axis=0, keepdims=True)
    cnt = jnp.concatenate(blocks, axis=0)
    counts_i = prefix.astype(jnp.int32)
    padded_i = ((counts_i + rb - 1) // rb) * rb
    padded_f = padded_i.astype(jnp.float32)
    triu = (lax.broadcasted_iota(jnp.int32, (n_e, n_e), 0)
            < lax.broadcasted_iota(jnp.int32, (n_e, n_e), 1)).astype(jnp.float32)
    aligned_f = jnp.dot(padded_f, triu, preferred_element_type=jnp.float32)
    base = aligned_f + cnt
    s0_ref[...] = jnp.sum(jnp.where(sel1, base, 0.0), axis=1,
                          keepdims=True).astype(jnp.int32)
    s1_ref[...] = jnp.sum(jnp.where(sel2, base, 0.0), axis=1,
                          keepdims=True).astype(jnp.int32)
    w0_ref[...] = m1 / denom
    w1_ref[...] = m2 / denom

    nbr = be_ref.shape[0]
    srow = lax.broadcasted_iota(jnp.int32, (nbr, n_e), 0) * rb
    al_b = jnp.broadcast_to(aligned_f.astype(jnp.int32), (nbr, n_e))
    pad_b = jnp.broadcast_to(padded_i, (nbr, n_e))
    eidx = lax.broadcasted_iota(jnp.int32, (nbr, n_e), 1)
    mask = (al_b <= srow) & (srow < al_b + pad_b)
    be_ref[...] = jnp.sum(jnp.where(mask, eidx, 0), axis=1, keepdims=True)
    act_ref[...] = jnp.sum(mask.astype(jnp.int32), axis=1, keepdims=True)


def _shared_body(x_ref, sg_ref, su_ref, sd_ref, out_ref):
    x = x_ref[...]
    g = jnp.dot(x, sg_ref[...].T, preferred_element_type=jnp.float32)
    u = jnp.dot(x, su_ref[...].T, preferred_element_type=jnp.float32)
    h = _silu(g) * u
    out_ref[...] = jnp.dot(h, sd_ref[...].T, preferred_element_type=jnp.float32)


def _ragged_body(be_ref, act_ref, xs_ref, w1_ref, w3_ref, w2_ref, y_ref):
    i = pl.program_id(0)

    @pl.when(act_ref[i] != 0)
    def _():
        xb = xs_ref[...]
        g = jnp.dot(xb, w1_ref[0].T, preferred_element_type=jnp.float32)
        u = jnp.dot(xb, w3_ref[0].T, preferred_element_type=jnp.float32)
        h = _silu(g) * u
        y_ref[...] = jnp.dot(h, w2_ref[0].T,
                             preferred_element_type=jnp.float32)


def _dispatch_sc(x, slot0, slot1):
    tpw = _T // _NW
    mesh = plsc.VectorSubcoreMesh(core_axis_name="c", subcore_axis_name="s")

    @functools.partial(
        pl.kernel,
        out_type=jax.ShapeDtypeStruct((_RSLOTS, _D), jnp.float32),
        mesh=mesh,
        compiler_params=pltpu.CompilerParams(needs_layout_passes=False),
        scratch_types=[
            pltpu.VMEM((tpw,), jnp.int32),
            pltpu.VMEM((tpw,), jnp.int32),
            pltpu.VMEM((tpw, _D), jnp.float32),
            pltpu.SemaphoreType.DMA,
            pltpu.SemaphoreType.DMA,
        ],
    )
    def k(s0_hbm, s1_hbm, x_hbm, xs_hbm, idx0_v, idx1_v, rows_v, sem0, sem1):
        wid = lax.axis_index("s") * 2 + lax.axis_index("c")
        t0 = wid * tpw
        pltpu.sync_copy(s0_hbm.at[pl.ds(t0, tpw)], idx0_v)
        pltpu.sync_copy(s1_hbm.at[pl.ds(t0, tpw)], idx1_v)
        pltpu.sync_copy(x_hbm.at[pl.ds(t0, tpw)], rows_v)
        c0 = pltpu.async_copy(rows_v, xs_hbm.at[idx0_v], sem0)
        c1 = pltpu.async_copy(rows_v, xs_hbm.at[idx1_v], sem1)
        c0.wait()
        c1.wait()

    return k(slot0, slot1, x)


def _combine_sc(slot0, slot1, w0a, w1a, y_sorted, shared):
    tpw = _T // _NW
    ck = 16
    mesh = plsc.VectorSubcoreMesh(core_axis_name="c", subcore_axis_name="s")

    @functools.partial(
        pl.kernel,
        out_type=jax.ShapeDtypeStruct((_T, _D), jnp.float32),
        mesh=mesh,
        compiler_params=pltpu.CompilerParams(needs_layout_passes=False),
        scratch_types=[
            pltpu.VMEM((tpw,), jnp.int32),
            pltpu.VMEM((tpw,), jnp.int32),
            pltpu.VMEM((tpw,), jnp.float32),
            pltpu.VMEM((tpw,), jnp.float32),
            pltpu.VMEM((ck, _D), jnp.float32),
            pltpu.VMEM((ck, _D), jnp.float32),
            pltpu.VMEM((ck, _D), jnp.float32),
            pltpu.VMEM((ck, _D), jnp.float32),
            pltpu.VMEM((ck, _D), jnp.float32),
            pltpu.VMEM((ck, _D), jnp.float32),
            pltpu.SemaphoreType.DMA,
            pltpu.SemaphoreType.DMA,
            pltpu.SemaphoreType.DMA,
            pltpu.SemaphoreType.DMA,
            pltpu.SemaphoreType.DMA,
            pltpu.SemaphoreType.DMA,
            pltpu.SemaphoreType.DMA,
            pltpu.SemaphoreType.DMA,
        ],
    )
    def k(s0_hbm, s1_hbm, w0_hbm, w1_hbm, y_hbm, sh_hbm, out_hbm,
          s0_v, s1_v, w0_v, w1_v, r0a_v, r1a_v, r0b_v, r1b_v, acca_v, accb_v,
          sa0, sa1, sb0, sb1, sha, shb, soa, sob):
        wid = lax.axis_index("s") * 2 + lax.axis_index("c")
        t0 = wid * tpw
        zeros = jnp.zeros((16,), jnp.int32)
        pltpu.sync_copy(s0_hbm.at[pl.ds(t0, tpw)], s0_v)
        pltpu.sync_copy(s1_hbm.at[pl.ds(t0, tpw)], s1_v)
        pltpu.sync_copy(w0_hbm.at[pl.ds(t0, tpw)], w0_v)
        pltpu.sync_copy(w1_hbm.at[pl.ds(t0, tpw)], w1_v)
        bufs = ((r0a_v, r1a_v, sa0, sa1), (r0b_v, r1b_v, sb0, sb1))
        accs = ((acca_v, sha, soa), (accb_v, shb, sob))
        nch = tpw // ck

        def fire(chn):
            r0, r1, g0, g1 = bufs[chn % 2]
            d0 = pltpu.async_copy(y_hbm.at[s0_v.at[pl.ds(chn * ck, ck)]],
                                  r0, g0)
            d1 = pltpu.async_copy(y_hbm.at[s1_v.at[pl.ds(chn * ck, ck)]],
                                  r1, g1)
            return d0, d1

        def fire_sh(chn):
            acc, ssem, _ = accs[chn % 2]
            return pltpu.async_copy(sh_hbm.at[pl.ds(t0 + chn * ck, ck)],
                                    acc, ssem)

        pend = fire(0)
        pend_sh = [fire_sh(0), None]
        pend_out = [None, None]
        for chn in range(nch):
            sl2 = (chn + 1) % 2
            if chn + 1 < nch:
                if pend_out[sl2] is not None:
                    pend_out[sl2].wait()
                    pend_out[sl2] = None
                nxt = fire(chn + 1)
                pend_sh[sl2] = fire_sh(chn + 1)
            else:
                nxt = None
            pend[0].wait()
            pend[1].wait()
            pend_sh[chn % 2].wait()
            r0, r1 = bufs[chn % 2][0], bufs[chn % 2][1]
            acc_v = accs[chn % 2][0]
            tb = t0 + chn * ck

            def _tok(i, carry, *, _chn=chn, _r0=r0, _r1=r1, _acc=acc_v):
                w0 = plsc.load_gather(w0_v, [zeros + (_chn * ck + i)])
                w1v = plsc.load_gather(w1_v, [zeros + (_chn * ck + i)])
                for j in range(_D // 16):
                    sl = pl.ds(j * 16, 16)
                    _acc[i, sl] = (_acc[i, sl] + w0 * _r0[i, sl]
                                   + w1v * _r1[i, sl])
                return carry
            lax.fori_loop(0, ck, _tok, 0)
            pend_out[chn % 2] = pltpu.async_copy(
                acc_v, out_hbm.at[pl.ds(tb, ck)], accs[chn % 2][2])
            pend = nxt
        for po in pend_out:
            if po is not None:
                po.wait()

    return k(slot0, slot1, w0a, w1a, y_sorted, shared)


def kernel(hidden_states, gate_w, w1, w2, w3, sg, su, sd):
    b, s, d = hidden_states.shape
    x = hidden_states.reshape(-1, d).astype(jnp.float32)
    t = x.shape[0]
    n_exp, d_ff, _ = w1.shape
    sf = sg.shape[0]
    n_tb = t // _TB

    slot0, slot1, w0a, w1a, be, act = pl.pallas_call(
        functools.partial(_route_body, rb=_RB),
        in_specs=[
            pl.BlockSpec((t, d), lambda: (0, 0)),
            pl.BlockSpec(gate_w.shape, lambda: (0, 0)),
        ],
        out_specs=[
            pl.BlockSpec((t, 1), lambda: (0, 0)),
            pl.BlockSpec((t, 1), lambda: (0, 0)),
            pl.BlockSpec((t, 1), lambda: (0, 0)),
            pl.BlockSpec((t, 1), lambda: (0, 0)),
            pl.BlockSpec((_NRB, 1), lambda: (0, 0)),
            pl.BlockSpec((_NRB, 1), lambda: (0, 0)),
        ],
        out_shape=[
            jax.ShapeDtypeStruct((t, 1), jnp.int32),
            jax.ShapeDtypeStruct((t, 1), jnp.int32),
            jax.ShapeDtypeStruct((t, 1), jnp.float32),
            jax.ShapeDtypeStruct((t, 1), jnp.float32),
            jax.ShapeDtypeStruct((_NRB, 1), jnp.int32),
            jax.ShapeDtypeStruct((_NRB, 1), jnp.int32),
        ],
    )(x, gate_w)

    slot0 = slot0.reshape(-1)
    slot1 = slot1.reshape(-1)
    w0a = w0a.reshape(-1)
    w1a = w1a.reshape(-1)
    be_flat = be.reshape(-1)

    x_sorted = _dispatch_sc(x, slot0, slot1)

    shared = pl.pallas_call(
        _shared_body,
        grid=(n_tb,),
        in_specs=[
            pl.BlockSpec((_TB, d), lambda i: (i, 0)),
            pl.BlockSpec((sf, d), lambda i: (0, 0)),
            pl.BlockSpec((sf, d), lambda i: (0, 0)),
            pl.BlockSpec((d, sf), lambda i: (0, 0)),
        ],
        out_specs=pl.BlockSpec((_TB, d), lambda i: (i, 0)),
        out_shape=jax.ShapeDtypeStruct((t, d), jnp.float32),
    )(x, sg, su, sd)

    y_sorted = pl.pallas_call(
        _ragged_body,
        grid_spec=pltpu.PrefetchScalarGridSpec(
            num_scalar_prefetch=2,
            grid=(_NRB,),
            in_specs=[
                pl.BlockSpec((_RB, d), lambda i, be_r, a_r: (i, 0)),
                pl.BlockSpec((1, d_ff, d),
                             lambda i, be_r, a_r: (be_r[i], 0, 0)),
                pl.BlockSpec((1, d_ff, d),
                             lambda i, be_r, a_r: (be_r[i], 0, 0)),
                pl.BlockSpec((1, d, d_ff),
                             lambda i, be_r, a_r: (be_r[i], 0, 0)),
            ],
            out_specs=pl.BlockSpec((_RB, d), lambda i, be_r, a_r: (i, 0)),
        ),
        out_shape=jax.ShapeDtypeStruct((_RSLOTS, d), jnp.float32),
    )(be_flat, act.reshape(-1), x_sorted, w1, w3, w2)

    out = _combine_sc(slot0, slot1, w0a, w1a, y_sorted, shared)
    return out.reshape(b, s, d).astype(hidden_states.dtype)
